# baseline (speedup 1.0000x reference)
import functools

import jax
import jax.numpy as jnp
from jax import lax
from jax.experimental import pallas as pl
from jax.experimental.pallas import tpu as pltpu
from jax.experimental.pallas import tpu_sc as plsc

_NC = 2
_NS = 16
_NW = _NC * _NS
_LANES = 16

_CHUNK = 32

_PARAMS = pltpu.CompilerParams(use_tc_tiling_on_sc=False,
                               needs_layout_passes=False,
                               disable_bounds_checks=True)


def _sc_kernel(B, L, D):
  nb_per_w = B // _NW
  n_chunks = nb_per_w // _CHUNK
  rows_per_chunk = _CHUNK * L
  n_idx_rows = rows_per_chunk // 128
  nk = D // _LANES

  mesh = plsc.VectorSubcoreMesh(core_axis_name="c", subcore_axis_name="s")

  @functools.partial(
      pl.kernel,
      out_type=jax.ShapeDtypeStruct((B * L,), jnp.float32),
      mesh=mesh,
      compiler_params=_PARAMS,
      scratch_types=[
          pltpu.VMEM((_CHUNK,), jnp.int32),
          pltpu.VMEM((n_idx_rows, 128), jnp.int32),
          pltpu.VMEM((_CHUNK, D), jnp.bfloat16),
          pltpu.VMEM((rows_per_chunk, D), jnp.bfloat16),
          pltpu.VMEM((rows_per_chunk,), jnp.float32),
          pltpu.SemaphoreType.DMA,
          pltpu.SemaphoreType.DMA,
      ],
  )
  def k(center_hbm, con_hbm, emb_u_hbm, emb_v_hbm, out_hbm,
        cidx_v, uidx_v, vrows_v, urows_v, res_v, sem_u, sem_v):
    wid = lax.axis_index("s") * _NC + lax.axis_index("c")
    lane = lax.iota(jnp.int32, _LANES)
    lane15 = lane == (_LANES - 1)
    perms = [(lane ^ (1 << p)).reshape(_LANES, 1) for p in range(4)]
    gdims = lax.GatherDimensionNumbers(
        offset_dims=(), collapsed_slice_dims=(0,), start_index_map=(0,))

    def lane_sum(x):
      for p in perms:
        x = x + lax.gather(x, p, gdims, (1,),
                           mode=lax.GatherScatterMode.PROMISE_IN_BOUNDS)
      return x

    def row_f32(ref, r):
      out = []
      for kk in range(nk // 2):
        ab = ref[r, pl.ds(kk * 2 * _LANES, 2 * _LANES)]
        out.extend(plsc.unpack(ab, format=plsc.PackFormat.INTERLEAVED))
      return out

    def chunk_body(c, _):
      b_base = wid * nb_per_w + c * _CHUNK
      pltpu.sync_copy(center_hbm.at[pl.ds(b_base, _CHUNK)], cidx_v)
      for j in range(n_idx_rows):
        pltpu.sync_copy(con_hbm.at[pl.ds(b_base * L + j * 128, 128)],
                        uidx_v.at[j])
      cp_v = pltpu.async_copy(emb_v_hbm.at[cidx_v], vrows_v, sem_v)
      for j in range(n_idx_rows):
        pltpu.async_copy(emb_u_hbm.at[uidx_v.at[j]],
                         urows_v.at[pl.ds(j * 128, 128)], sem_u)
      cp_v.wait()
      pltpu.make_async_copy(
          emb_u_hbm.at[pl.ds(0, rows_per_chunk), :], urows_v, sem_u).wait()

      def dot_body(b, _):
        vr = row_f32(vrows_v, b)
        for l in range(L):
          row = b * L + l
          ur = row_f32(urows_v, row)
          acc = ur[0] * vr[0]
          for kk in range(1, nk):
            acc += ur[kk] * vr[kk]
          tot = lane_sum(acc)
          plsc.store_scatter(
              res_v, [jnp.full((_LANES,), row, jnp.int32)], tot, mask=lane15)
        return ()

      lax.fori_loop(0, _CHUNK, dot_body, ())
      pltpu.sync_copy(res_v, out_hbm.at[pl.ds(b_base * L, rows_per_chunk)])
      return ()

    lax.fori_loop(0, n_chunks, chunk_body, ())

  return k


def kernel(center, con_neg, emb_u, emb_v):
  B, L = con_neg.shape
  V, D = emb_u.shape
  assert B % (_NW * _CHUNK) == 0 and (_CHUNK * L) % 128 == 0
  assert D % _LANES == 0
  center_flat = center.reshape(B).astype(jnp.int32)
  con_flat = con_neg.reshape(B * L).astype(jnp.int32)
  out = _sc_kernel(B, L, D)(center_flat, con_flat,
                            emb_u.astype(jnp.bfloat16),
                            emb_v.astype(jnp.bfloat16))
  return out.reshape(B, 1, L)


# speedup vs baseline: 1.2385x; 1.2385x over previous
import functools

import jax
import jax.numpy as jnp
from jax import lax
from jax.experimental import pallas as pl
from jax.experimental.pallas import tpu as pltpu
from jax.experimental.pallas import tpu_sc as plsc

_NC = 2
_NS = 16
_NW = _NC * _NS
_LANES = 16

_CHUNK = 32

_PARAMS = pltpu.CompilerParams(use_tc_tiling_on_sc=False,
                               needs_layout_passes=False,
                               disable_bounds_checks=True)


def _sc_kernel(B, L, D):
  nb_per_w = B // _NW
  n_chunks = nb_per_w // _CHUNK
  rows_per_chunk = _CHUNK * L
  n_idx_rows = rows_per_chunk // 128
  nk = D // _LANES

  mesh = plsc.VectorSubcoreMesh(core_axis_name="c", subcore_axis_name="s")

  @functools.partial(
      pl.kernel,
      out_type=jax.ShapeDtypeStruct((B * L,), jnp.float32),
      mesh=mesh,
      compiler_params=_PARAMS,
      scratch_types=[
          pltpu.VMEM((_CHUNK,), jnp.int32),
          pltpu.VMEM((n_idx_rows, 128), jnp.int32),
          pltpu.VMEM((_CHUNK, D), jnp.float32),
          pltpu.VMEM((rows_per_chunk, D), jnp.float32),
          pltpu.VMEM((rows_per_chunk,), jnp.float32),
          pltpu.SemaphoreType.DMA,
          pltpu.SemaphoreType.DMA,
      ],
  )
  def k(center_hbm, con_hbm, emb_u_hbm, emb_v_hbm, out_hbm,
        cidx_v, uidx_v, vrows_v, urows_v, res_v, sem_u, sem_v):
    wid = lax.axis_index("s") * _NC + lax.axis_index("c")
    lane = lax.iota(jnp.int32, _LANES)
    lane15 = lane == (_LANES - 1)
    perms = [(lane ^ (1 << p)).reshape(_LANES, 1) for p in range(4)]
    gdims = lax.GatherDimensionNumbers(
        offset_dims=(), collapsed_slice_dims=(0,), start_index_map=(0,))

    def lane_sum(x):
      for p in perms:
        x = x + lax.gather(x, p, gdims, (1,),
                           mode=lax.GatherScatterMode.PROMISE_IN_BOUNDS)
      return x

    def chunk_body(c, _):
      b_base = wid * nb_per_w + c * _CHUNK
      pltpu.sync_copy(center_hbm.at[pl.ds(b_base, _CHUNK)], cidx_v)
      for j in range(n_idx_rows):
        pltpu.sync_copy(con_hbm.at[pl.ds(b_base * L + j * 128, 128)],
                        uidx_v.at[j])
      cp_v = pltpu.async_copy(emb_v_hbm.at[cidx_v], vrows_v, sem_v)
      for j in range(n_idx_rows):
        pltpu.async_copy(emb_u_hbm.at[uidx_v.at[j]],
                         urows_v.at[pl.ds(j * 128, 128)], sem_u)
      cp_v.wait()
      pltpu.make_async_copy(
          emb_u_hbm.at[pl.ds(0, rows_per_chunk), :], urows_v, sem_u).wait()

      def dot_body(b, _):
        vr = [vrows_v[b, pl.ds(kk * _LANES, _LANES)] for kk in range(nk)]
        for l in range(L):
          row = b * L + l
          acc = urows_v[row, pl.ds(0, _LANES)] * vr[0]
          for kk in range(1, nk):
            acc += urows_v[row, pl.ds(kk * _LANES, _LANES)] * vr[kk]
          tot = lane_sum(acc)
          plsc.store_scatter(
              res_v, [jnp.full((_LANES,), row, jnp.int32)], tot, mask=lane15)
        return ()

      lax.fori_loop(0, _CHUNK, dot_body, ())
      pltpu.sync_copy(res_v, out_hbm.at[pl.ds(b_base * L, rows_per_chunk)])
      return ()

    lax.fori_loop(0, n_chunks, chunk_body, ())

  return k


def kernel(center, con_neg, emb_u, emb_v):
  B, L = con_neg.shape
  V, D = emb_u.shape
  assert B % (_NW * _CHUNK) == 0 and (_CHUNK * L) % 128 == 0
  assert D % _LANES == 0
  center_flat = center.reshape(B).astype(jnp.int32)
  con_flat = con_neg.reshape(B * L).astype(jnp.int32)
  out = _sc_kernel(B, L, D)(center_flat, con_flat, emb_u, emb_v)
  return out.reshape(B, 1, L)


# speedup vs baseline: 1.3726x; 1.1083x over previous
import functools

import jax
import jax.numpy as jnp
from jax import lax
from jax.experimental import pallas as pl
from jax.experimental.pallas import tpu as pltpu
from jax.experimental.pallas import tpu_sc as plsc

_NC = 2
_NS = 16
_NW = _NC * _NS
_LANES = 16
_LPAD = 2 * _LANES

_CHUNK = 32


def _sc_kernel(B, L, D):
  nb_per_w = B // _NW
  n_chunks = nb_per_w // _CHUNK
  rows_per_chunk = _CHUNK * L
  n_idx_rows = rows_per_chunk // 128
  nk = D // _LANES

  mesh = plsc.VectorSubcoreMesh(core_axis_name="c", subcore_axis_name="s")

  @functools.partial(
      pl.kernel,
      out_type=jax.ShapeDtypeStruct((B * _LPAD,), jnp.float32),
      mesh=mesh,
      compiler_params=pltpu.CompilerParams(use_tc_tiling_on_sc=False),
      scratch_types=[
          pltpu.VMEM((_CHUNK,), jnp.int32),
          pltpu.VMEM((n_idx_rows, 128), jnp.int32),
          pltpu.VMEM((_CHUNK, D), jnp.float32),
          pltpu.VMEM((rows_per_chunk, D), jnp.float32),
          pltpu.VMEM((_CHUNK * _LPAD,), jnp.float32),
          pltpu.SemaphoreType.DMA,
          pltpu.SemaphoreType.DMA,
      ],
  )
  def k(center_hbm, con_hbm, emb_u_hbm, emb_v_hbm, out_hbm,
        cidx_v, uidx_v, vrows_v, urows_v, res_v, sem_u, sem_v):
    wid = lax.axis_index("s") * _NC + lax.axis_index("c")
    lane = lax.iota(jnp.int32, _LANES)
    perms = [
        (lane ^ (1 << p)).reshape(_LANES, 1) for p in range(4)
    ]
    gdims = lax.GatherDimensionNumbers(
        offset_dims=(), collapsed_slice_dims=(0,), start_index_map=(0,))

    def lane_sum(x):
      for p in perms:
        x = x + lax.gather(x, p, gdims, (1,),
                           mode=lax.GatherScatterMode.PROMISE_IN_BOUNDS)
      return x

    def chunk_body(c, _):
      b_base = wid * nb_per_w + c * _CHUNK
      pltpu.sync_copy(center_hbm.at[pl.ds(b_base, _CHUNK)], cidx_v)
      for j in range(n_idx_rows):
        pltpu.sync_copy(con_hbm.at[pl.ds(b_base * L + j * 128, 128)],
                        uidx_v.at[j])
      cp_v = pltpu.async_copy(emb_v_hbm.at[cidx_v], vrows_v, sem_v)
      cps = [
          pltpu.async_copy(
              emb_u_hbm.at[uidx_v.at[j]],
              urows_v.at[pl.ds(j * 128, 128)], sem_u)
          for j in range(n_idx_rows)
      ]
      cp_v.wait()
      for cp in cps:
        cp.wait()

      def dot_body(b, _):
        vr = [vrows_v[b, pl.ds(k16 * _LANES, _LANES)] for k16 in range(nk)]

        def dot(l):
          row = b * L + l
          acc = urows_v[row, pl.ds(0, _LANES)] * vr[0]
          for k16 in range(1, nk):
            acc += urows_v[row, pl.ds(k16 * _LANES, _LANES)] * vr[k16]
          return lane_sum(acc)

        vec0 = dot(0)
        for l in range(1, _LANES):
          vec0 = jnp.where(lane == l, dot(l), vec0)
        res_v[pl.ds(b * _LPAD, _LANES)] = vec0
        vec1 = dot(_LANES)
        for l in range(_LANES + 1, L):
          vec1 = jnp.where(lane == (l - _LANES), dot(l), vec1)
        res_v[pl.ds(b * _LPAD + _LANES, _LANES)] = vec1
        return ()

      lax.fori_loop(0, _CHUNK, dot_body, ())
      pltpu.sync_copy(res_v,
                      out_hbm.at[pl.ds(b_base * _LPAD, _CHUNK * _LPAD)])
      return ()

    lax.fori_loop(0, n_chunks, chunk_body, ())

  return k


def kernel(center, con_neg, emb_u, emb_v):
  B, L = con_neg.shape
  V, D = emb_u.shape
  assert B % (_NW * _CHUNK) == 0 and (_CHUNK * L) % 128 == 0
  assert D % _LANES == 0 and _LANES < L <= _LPAD
  center_flat = center.reshape(B).astype(jnp.int32)
  con_flat = con_neg.reshape(B * L).astype(jnp.int32)
  out = _sc_kernel(B, L, D)(center_flat, con_flat, emb_u, emb_v)
  return out.reshape(B, _LPAD)[:, :L].reshape(B, 1, L)
